# Initial kernel scaffold; baseline (speedup 1.0000x reference)
import jax
import jax.numpy as jnp
from jax.experimental import pallas as pl


def kernel(x, edge_index, W_w, W_b, B_w, B_b):
    raise NotImplementedError("write your pallas kernel here")


# speedup vs baseline: 12.7535x; 12.7535x over previous
import functools

import jax
import jax.numpy as jnp
from jax import lax
from jax.experimental import pallas as pl
from jax.experimental.pallas import tpu as pltpu
from jax.experimental.pallas import tpu_sc as plsc

N = 4096
C = 64
H = 8
W = 8
E = 131072
F = H * W * C


NC = 2
NS = 16
NW = NC * NS
LANES = 16
ROWS_PB = 16
PASSES = N // (NW * ROWS_PB)
CHUNK_E = 8192
NCHUNK = E // CHUNK_E

def _build_adj_body(dst_hbm, src_hbm, a_out, dbuf, sbuf, acc):
    wid = lax.axis_index("s") * NC + lax.axis_index("c")
    zeros16 = jnp.zeros((LANES,), jnp.float32)
    ones16 = jnp.ones((LANES,), jnp.float32)

    def pass_body(p, _):
        base = (wid * PASSES + p) * ROWS_PB

        def zero_body(j, _):
            acc[pl.ds(j * LANES, LANES)] = zeros16
            return _

        lax.fori_loop(0, (ROWS_PB * N) // LANES, zero_body, 0)

        def chunk_body(ci, _):
            off = ci * CHUNK_E
            pltpu.sync_copy(dst_hbm.at[pl.ds(off, CHUNK_E)], dbuf)
            pltpu.sync_copy(src_hbm.at[pl.ds(off, CHUNK_E)], sbuf)

            def edge_body(i, _):
                dv = dbuf[pl.ds(i * LANES, LANES)]
                sv = sbuf[pl.ds(i * LANES, LANES)]
                rel = dv - base
                m = (rel >= 0) & (rel < ROWS_PB)
                relc = jnp.clip(rel, 0, ROWS_PB - 1)
                flat = relc * N + sv
                val = jnp.where(m, 1.0, 0.0).astype(jnp.float32)
                plsc.addupdate_scatter(acc, [flat], val)
                return _

            lax.fori_loop(0, CHUNK_E // LANES, edge_body, 0)
            return _

        lax.fori_loop(0, NCHUNK, chunk_body, 0)
        pltpu.sync_copy(acc, a_out.at[pl.ds(base * N, ROWS_PB * N)])
        return _

    lax.fori_loop(0, PASSES, pass_body, 0)


@functools.cache
def _get_build_adj():
    mesh = plsc.VectorSubcoreMesh(core_axis_name="c", subcore_axis_name="s")
    return pl.kernel(
        _build_adj_body,
        mesh=mesh,
        compiler_params=pltpu.CompilerParams(needs_layout_passes=False),
        out_type=jax.ShapeDtypeStruct((N * N,), jnp.float32),
        scratch_types=[
            pltpu.VMEM((CHUNK_E,), jnp.int32),
            pltpu.VMEM((CHUNK_E,), jnp.int32),
            pltpu.VMEM((ROWS_PB * N,), jnp.float32),
        ],
    )


BN = 32
TAPS = [(dh, dw) for dh in range(3) for dw in range(3)]


def _conv_body(x_ref, wb_ref, out_ref):
    xb = x_ref[...]
    xp = jnp.pad(xb, ((0, 0), (1, 1), (1, 1), (0, 0)))
    cols = [
        xp[:, dh:dh + H, dw:dw + W, :].reshape(BN * H * W, C)
        for (dh, dw) in TAPS
    ]
    xcol = jnp.concatenate(cols, axis=1)
    out_ref[...] = jnp.dot(
        xcol, wb_ref[...], preferred_element_type=jnp.float32
    )


def _convs(x2, wbflat):
    return pl.pallas_call(
        _conv_body,
        grid=(N // BN,),
        in_specs=[
            pl.BlockSpec((BN, H, W, C), lambda i: (i, 0, 0, 0)),
            pl.BlockSpec((9 * C, 2 * C), lambda i: (0, 0)),
        ],
        out_specs=pl.BlockSpec((BN * H * W, 2 * C), lambda i: (i, 0)),
        out_shape=jax.ShapeDtypeStruct((N * H * W, 2 * C), jnp.float32),
    )(x2, wbflat)


BM = 256
BK = 512
NK = N // BK


def _agg_body(a_ref, u_ref, yw_ref, wb_ref, bb_ref, y_ref, acc_ref, deg_ref):
    k = pl.program_id(1)

    @pl.when(k == 0)
    def _():
        acc_ref[...] = jnp.zeros_like(acc_ref)
        deg_ref[...] = jnp.zeros_like(deg_ref)

    ab = a_ref[...]
    acc_ref[...] += jnp.dot(ab, u_ref[...], preferred_element_type=jnp.float32)
    deg_ref[...] += jnp.sum(ab, axis=1, keepdims=True)

    @pl.when(k == NK - 1)
    def _():
        deg = jnp.maximum(deg_ref[...], 1.0)
        y = yw_ref[...] + wb_ref[...] + acc_ref[...] / deg + bb_ref[...]
        y_ref[...] = jnp.maximum(y, 0.0)


def _aggregate(a, u, yw, wb_full, bb_full):
    return pl.pallas_call(
        _agg_body,
        grid=(N // BM, NK),
        in_specs=[
            pl.BlockSpec((BM, BK), lambda i, k: (i, k)),
            pl.BlockSpec((BK, F), lambda i, k: (k, 0)),
            pl.BlockSpec((BM, F), lambda i, k: (i, 0)),
            pl.BlockSpec((1, F), lambda i, k: (0, 0)),
            pl.BlockSpec((1, F), lambda i, k: (0, 0)),
        ],
        out_specs=pl.BlockSpec((BM, F), lambda i, k: (i, 0)),
        out_shape=jax.ShapeDtypeStruct((N, F), jnp.float32),
        scratch_shapes=[
            pltpu.VMEM((BM, F), jnp.float32),
            pltpu.VMEM((BM, 1), jnp.float32),
        ],
    )(a, u, yw, wb_full, bb_full)


@jax.jit
def kernel(x, edge_index, W_w, W_b, B_w, B_b):
    src = edge_index[0]
    dst = edge_index[1]

    x2 = jnp.transpose(x, (0, 2, 3, 1))

    wb = jnp.stack(
        [
            jnp.concatenate(
                [W_w[:, :, dh, dw].T, B_w[:, :, dh, dw].T], axis=1
            )
            for (dh, dw) in TAPS
        ]
    )
    wbflat = wb.reshape(9 * C, 2 * C)

    wb_full = jnp.tile(W_b, H * W)[None, :]
    bb_full = jnp.tile(B_b, H * W)[None, :]

    a = _get_build_adj()(dst, src).reshape(N, N)
    z = _convs(x2, wbflat).reshape(N, H * W, 2 * C)
    yw = z[:, :, :C].reshape(N, F)
    u = z[:, :, C:].reshape(N, F)
    y = _aggregate(a, u, yw, wb_full, bb_full)

    return jnp.transpose(y.reshape(N, H, W, C), (0, 3, 1, 2))


# speedup vs baseline: 16.4665x; 1.2911x over previous
import functools

import jax
import jax.numpy as jnp
from jax import lax
from jax.experimental import pallas as pl
from jax.experimental.pallas import tpu as pltpu
from jax.experimental.pallas import tpu_sc as plsc

N = 4096
C = 64
H = 8
W = 8
E = 131072
F = H * W * C


NC = 2
NS = 16
NW = NC * NS
LANES = 16
ROWS_PB = 16
PASSES = N // (NW * ROWS_PB)
CHUNK_E = 8192
NCHUNK = E // CHUNK_E
UNROLL = 8

def _build_adj_body(dst_hbm, src_hbm, a_out, dbuf, sbuf, acc, dsem, ssem):
    wid = lax.axis_index("s") * NC + lax.axis_index("c")
    zeros16 = jnp.zeros((LANES,), jnp.float32)

    def start_chunk(ci, slot):
        off = ci * CHUNK_E
        pltpu.async_copy(
            dst_hbm.at[pl.ds(off, CHUNK_E)],
            dbuf.at[pl.ds(slot * CHUNK_E, CHUNK_E)],
            dsem,
        )
        pltpu.async_copy(
            src_hbm.at[pl.ds(off, CHUNK_E)],
            sbuf.at[pl.ds(slot * CHUNK_E, CHUNK_E)],
            ssem,
        )

    def wait_chunk(slot):
        pltpu.make_async_copy(
            dst_hbm.at[pl.ds(0, CHUNK_E)],
            dbuf.at[pl.ds(slot * CHUNK_E, CHUNK_E)],
            dsem,
        ).wait()
        pltpu.make_async_copy(
            src_hbm.at[pl.ds(0, CHUNK_E)],
            sbuf.at[pl.ds(slot * CHUNK_E, CHUNK_E)],
            ssem,
        ).wait()

    def process_group(gbase, base):
        for uu in range(UNROLL):
            o = gbase + uu * LANES
            dv = dbuf[pl.ds(o, LANES)]
            sv = sbuf[pl.ds(o, LANES)]
            rel = dv - base
            m = plsc.bitcast(rel, jnp.uint32) < jnp.uint32(ROWS_PB)
            flat = ((rel & (ROWS_PB - 1)) << 12) | sv
            val = m.astype(jnp.float32)
            plsc.addupdate_scatter(acc, [flat], val)

    def pass_body(p, _):
        base = (wid * PASSES + p) * ROWS_PB

        def zero_body(j, _):
            acc[pl.ds(j * LANES, LANES)] = zeros16
            return _

        lax.fori_loop(0, (ROWS_PB * N) // LANES, zero_body, 0)

        start_chunk(0, 0)
        GE = LANES * UNROLL

        def chunk_body(ci, _):
            slot = lax.rem(ci, 2)
            wait_chunk(slot)

            @pl.when(ci + 1 < NCHUNK)
            def _():
                start_chunk(ci + 1, 1 - slot)

            sbase = slot * CHUNK_E

            def edge_body(i, _):
                process_group(sbase + i * GE, base)
                return _

            lax.fori_loop(0, CHUNK_E // GE, edge_body, 0)
            return 0

        lax.fori_loop(0, NCHUNK, chunk_body, 0)
        pltpu.sync_copy(acc, a_out.at[pl.ds(base * N, ROWS_PB * N)])
        return 0

    lax.fori_loop(0, PASSES, pass_body, 0)


@functools.cache
def _get_build_adj():
    mesh = plsc.VectorSubcoreMesh(core_axis_name="c", subcore_axis_name="s")
    return pl.kernel(
        _build_adj_body,
        mesh=mesh,
        compiler_params=pltpu.CompilerParams(needs_layout_passes=False),
        out_type=jax.ShapeDtypeStruct((N * N,), jnp.float32),
        scratch_types=[
            pltpu.VMEM((2 * CHUNK_E,), jnp.int32),
            pltpu.VMEM((2 * CHUNK_E,), jnp.int32),
            pltpu.VMEM((ROWS_PB * N,), jnp.float32),
            pltpu.SemaphoreType.DMA,
            pltpu.SemaphoreType.DMA,
        ],
    )


BN = 32
TAPS = [(dh, dw) for dh in range(3) for dw in range(3)]


def _conv_body(x_ref, wb_ref, out_ref):
    xb = x_ref[...]
    xp = jnp.pad(xb, ((0, 0), (1, 1), (1, 1), (0, 0)))
    cols = [
        xp[:, dh:dh + H, dw:dw + W, :].reshape(BN * H * W, C)
        for (dh, dw) in TAPS
    ]
    xcol = jnp.concatenate(cols, axis=1)
    out_ref[...] = jnp.dot(
        xcol, wb_ref[...], preferred_element_type=jnp.float32
    )


def _convs(x2, wbflat):
    return pl.pallas_call(
        _conv_body,
        grid=(N // BN,),
        in_specs=[
            pl.BlockSpec((BN, H, W, C), lambda i: (i, 0, 0, 0)),
            pl.BlockSpec((9 * C, 2 * C), lambda i: (0, 0)),
        ],
        out_specs=pl.BlockSpec((BN * H * W, 2 * C), lambda i: (i, 0)),
        out_shape=jax.ShapeDtypeStruct((N * H * W, 2 * C), jnp.float32),
    )(x2, wbflat)


BM = 256
BK = 512
NK = N // BK


def _agg_body(a_ref, u_ref, yw_ref, wb_ref, bb_ref, y_ref, acc_ref, deg_ref):
    k = pl.program_id(1)

    @pl.when(k == 0)
    def _():
        acc_ref[...] = jnp.zeros_like(acc_ref)
        deg_ref[...] = jnp.zeros_like(deg_ref)

    ab = a_ref[...]
    acc_ref[...] += jnp.dot(ab, u_ref[...], preferred_element_type=jnp.float32)
    deg_ref[...] += jnp.sum(ab, axis=1, keepdims=True)

    @pl.when(k == NK - 1)
    def _():
        deg = jnp.maximum(deg_ref[...], 1.0)
        y = yw_ref[...] + wb_ref[...] + acc_ref[...] / deg + bb_ref[...]
        y_ref[...] = jnp.maximum(y, 0.0)


def _aggregate(a, u, yw, wb_full, bb_full):
    return pl.pallas_call(
        _agg_body,
        grid=(N // BM, NK),
        in_specs=[
            pl.BlockSpec((BM, BK), lambda i, k: (i, k)),
            pl.BlockSpec((BK, F), lambda i, k: (k, 0)),
            pl.BlockSpec((BM, F), lambda i, k: (i, 0)),
            pl.BlockSpec((1, F), lambda i, k: (0, 0)),
            pl.BlockSpec((1, F), lambda i, k: (0, 0)),
        ],
        out_specs=pl.BlockSpec((BM, F), lambda i, k: (i, 0)),
        out_shape=jax.ShapeDtypeStruct((N, F), jnp.float32),
        scratch_shapes=[
            pltpu.VMEM((BM, F), jnp.float32),
            pltpu.VMEM((BM, 1), jnp.float32),
        ],
    )(a, u, yw, wb_full, bb_full)


@jax.jit
def kernel(x, edge_index, W_w, W_b, B_w, B_b):
    src = edge_index[0]
    dst = edge_index[1]

    x2 = jnp.transpose(x, (0, 2, 3, 1))

    wb = jnp.stack(
        [
            jnp.concatenate(
                [W_w[:, :, dh, dw].T, B_w[:, :, dh, dw].T], axis=1
            )
            for (dh, dw) in TAPS
        ]
    )
    wbflat = wb.reshape(9 * C, 2 * C)

    wb_full = jnp.tile(W_b, H * W)[None, :]
    bb_full = jnp.tile(B_b, H * W)[None, :]

    a = _get_build_adj()(dst, src).reshape(N, N)
    z = _convs(x2, wbflat).reshape(N, H * W, 2 * C)
    yw = z[:, :, :C].reshape(N, F)
    u = z[:, :, C:].reshape(N, F)
    y = _aggregate(a, u, yw, wb_full, bb_full)

    return jnp.transpose(y.reshape(N, H, W, C), (0, 3, 1, 2))
